# baseline (speedup 1.0000x reference)
import functools

import jax
import jax.numpy as jnp
from jax import lax
from jax.experimental import pallas as pl
from jax.experimental.pallas import tpu as pltpu
from jax.experimental.pallas import tpu_sc as plsc

_NC = 2
_NS = 16
_NW = _NC * _NS
_L = 16


def _make_sc_partials(B, H, W, half):
    HW = H * W
    nw = _NS
    assert B % (2 * 2 * nw) == 0 and HW % _L == 0
    bpw = (B // 2) // nw
    n_steps = HW // _L
    mesh = plsc.VectorSubcoreMesh(core_axis_name="c", subcore_axis_name="s",
                                  num_cores=1)

    @functools.partial(
        pl.kernel,
        mesh=mesh,
        compiler_params=pltpu.CompilerParams(needs_layout_passes=False),
        out_type=[
            jax.ShapeDtypeStruct((_NS, _L), jnp.float32),
            jax.ShapeDtypeStruct((_NS, _L), jnp.float32),
        ],
        scratch_types=[
            pltpu.VMEM((3, HW), jnp.float32),
            pltpu.VMEM((3, HW), jnp.float32),
            pltpu.VMEM((HW,), jnp.float32),
            pltpu.VMEM((HW,), jnp.float32),
            pltpu.VMEM((HW,), jnp.float32),
            pltpu.VMEM((HW,), jnp.float32),
            pltpu.VMEM((_L,), jnp.float32),
            pltpu.VMEM((_L,), jnp.float32),
            pltpu.SemaphoreType.DMA,
            pltpu.SemaphoreType.DMA,
        ],
    )
    def sc_kernel(curr_hbm, roi_hbm, mask_hbm, loss_out, cnt_out,
                  roi_v0, roi_v1, curr_v0, curr_v1, mask_v0, mask_v1,
                  loss_v, cnt_v, sem0, sem1):
        wid = lax.axis_index("s")
        base = half * (B // 2) + wid * bpw
        bufs = ((roi_v0, curr_v0, mask_v0, sem0),
                (roi_v1, curr_v1, mask_v1, sem1))

        def fire(b, k):
            roi_v, curr_v, mask_v, sem = bufs[k]
            pltpu.make_async_copy(curr_hbm.at[b], curr_v, sem).start()
            pltpu.make_async_copy(roi_hbm.at[b], roi_v, sem).start()
            pltpu.make_async_copy(mask_hbm.at[b], mask_v, sem).start()

        def drain(b, k):
            roi_v, curr_v, mask_v, sem = bufs[k]
            pltpu.make_async_copy(curr_hbm.at[b], curr_v, sem).wait()
            pltpu.make_async_copy(roi_hbm.at[b], roi_v, sem).wait()
            pltpu.make_async_copy(mask_hbm.at[b], mask_v, sem).wait()

        def compute(k, accs):
            roi_v, curr_v, mask_v, _ = bufs[k]

            def step(j, accs2):
                lacc, cacc = accs2
                sl = pl.ds(j * _L, _L)
                yf = roi_v[0, sl]
                xf = roi_v[1, sl]
                t = roi_v[2, sl]
                y = yf.astype(jnp.int32)
                x = xf.astype(jnp.int32)
                valid = (lax.bitcast_convert_type(y, jnp.uint32) < H) & (
                    lax.bitcast_convert_type(x, jnp.uint32) < W)
                flat = jnp.minimum(jnp.maximum(y * W + x, 0), HW - 1)
                c = plsc.load_gather(curr_v, [flat])
                m = plsc.load_gather(mask_v, [flat])
                d = c - t
                ad = jnp.abs(d)
                loss = jnp.where(ad < 1.0, 0.5 * d * d, ad - 0.5)
                w = jnp.where(valid, m, 0.0)
                return (lacc + loss * w, cacc + w)

            return lax.fori_loop(0, n_steps, step, accs, unroll=8)

        fire(base, 0)

        def pair_body(ip, accs):
            for k in (0, 1):
                i = 2 * ip + k
                b = base + i

                @pl.when(i + 1 < bpw)
                def _():
                    fire(b + 1, 1 - k)

                drain(b, k)
                accs = compute(k, accs)
            return accs

        zero = jnp.zeros((_L,), jnp.float32)
        lacc, cacc = lax.fori_loop(0, bpw // 2, pair_body, (zero, zero))
        loss_v[...] = lacc
        cnt_v[...] = cacc
        pltpu.sync_copy(loss_v, loss_out.at[wid])
        pltpu.sync_copy(cnt_v, cnt_out.at[wid])

    return sc_kernel


def _finish(loss_ref, cnt_ref, out_ref):
    ls = jnp.sum(loss_ref[...])
    nv = jnp.sum(cnt_ref[...])
    out_ref[...] = (ls / (nv + 1e-6)).reshape(1, 1)


def kernel(curr_heightmap, new_roi, mask):
    B, _, H, W = curr_heightmap.shape
    HW = H * W
    curr2 = curr_heightmap.reshape(B, HW)
    roi2 = new_roi.reshape(B, 3, HW)
    mask2 = mask.reshape(B, HW)
    loss_p0, cnt_p0 = _make_sc_partials(B, H, W, 0)(curr2, roi2, mask2)
    loss_p1, cnt_p1 = _make_sc_partials(B, H, W, 1)(curr2, roi2, mask2)
    out = pl.pallas_call(
        _finish,
        out_shape=jax.ShapeDtypeStruct((1, 1), jnp.float32),
    )(jnp.concatenate([loss_p0, loss_p1]), jnp.concatenate([cnt_p0, cnt_p1]))
    return out[0, 0]


# speedup vs baseline: 1.1444x; 1.1444x over previous
import functools

import jax
import jax.numpy as jnp
from jax import lax
from jax.experimental import pallas as pl
from jax.experimental.pallas import tpu as pltpu
from jax.experimental.pallas import tpu_sc as plsc

_NC = 2
_NS = 16
_NW = _NC * _NS
_L = 16


def _make_sc_partials(B, H, W):
    HW = H * W
    assert B % (2 * _NW) == 0 and HW % _L == 0
    bpw = B // _NW
    n_steps = HW // _L
    mesh = plsc.VectorSubcoreMesh(core_axis_name="c", subcore_axis_name="s")

    @functools.partial(
        pl.kernel,
        mesh=mesh,
        compiler_params=pltpu.CompilerParams(needs_layout_passes=False),
        out_type=[
            jax.ShapeDtypeStruct((_NW, _L), jnp.float32),
            jax.ShapeDtypeStruct((_NW, _L), jnp.float32),
        ],
        scratch_types=[
            pltpu.VMEM((3, HW), jnp.float32),
            pltpu.VMEM((3, HW), jnp.float32),
            pltpu.VMEM((HW,), jnp.float32),
            pltpu.VMEM((HW,), jnp.float32),
            pltpu.VMEM((HW,), jnp.float32),
            pltpu.VMEM((HW,), jnp.float32),
            pltpu.VMEM((_L,), jnp.float32),
            pltpu.VMEM((_L,), jnp.float32),
            pltpu.SemaphoreType.DMA,
            pltpu.SemaphoreType.DMA,
        ],
    )
    def sc_kernel(curr_hbm, roi_hbm, mask_hbm, loss_out, cnt_out,
                  roi_v0, roi_v1, curr_v0, curr_v1, mask_v0, mask_v1,
                  loss_v, cnt_v, sem0, sem1):
        wid = lax.axis_index("s") * _NC + lax.axis_index("c")
        base = wid * bpw
        bufs = ((roi_v0, curr_v0, mask_v0, sem0),
                (roi_v1, curr_v1, mask_v1, sem1))

        def fire(b, k):
            roi_v, curr_v, mask_v, sem = bufs[k]
            pltpu.make_async_copy(curr_hbm.at[b], curr_v, sem).start()
            pltpu.make_async_copy(roi_hbm.at[b], roi_v, sem).start()
            pltpu.make_async_copy(mask_hbm.at[b], mask_v, sem).start()

        def drain(b, k):
            roi_v, curr_v, mask_v, sem = bufs[k]
            pltpu.make_async_copy(curr_hbm.at[b], curr_v, sem).wait()
            pltpu.make_async_copy(roi_hbm.at[b], roi_v, sem).wait()
            pltpu.make_async_copy(mask_hbm.at[b], mask_v, sem).wait()

        def compute(k, accs):
            roi_v, curr_v, mask_v, _ = bufs[k]

            def step(j, accs2):
                lacc, cacc = accs2
                sl = pl.ds(j * _L, _L)
                yf = roi_v[0, sl]
                xf = roi_v[1, sl]
                t = roi_v[2, sl]
                y = yf.astype(jnp.int32)
                x = xf.astype(jnp.int32)
                valid = (lax.bitcast_convert_type(y, jnp.uint32) < H) & (
                    lax.bitcast_convert_type(x, jnp.uint32) < W)
                flat = jnp.minimum(jnp.maximum(y * W + x, 0), HW - 1)
                c = plsc.load_gather(curr_v, [flat])
                m = plsc.load_gather(mask_v, [flat])
                d = c - t
                ad = jnp.abs(d)
                loss = jnp.where(ad < 1.0, 0.5 * d * d, ad - 0.5)
                w = jnp.where(valid, m, 0.0)
                return (lacc + loss * w, cacc + w)

            return lax.fori_loop(0, n_steps, step, accs, unroll=8)

        fire(base, 0)

        def pair_body(ip, accs):
            for k in (0, 1):
                i = 2 * ip + k
                b = base + i

                @pl.when(i + 1 < bpw)
                def _():
                    fire(b + 1, 1 - k)

                drain(b, k)
                accs = compute(k, accs)
            return accs

        zero = jnp.zeros((_L,), jnp.float32)
        lacc, cacc = lax.fori_loop(0, bpw // 2, pair_body, (zero, zero))
        loss_v[...] = lacc
        cnt_v[...] = cacc
        pltpu.sync_copy(loss_v, loss_out.at[wid])
        pltpu.sync_copy(cnt_v, cnt_out.at[wid])

    return sc_kernel


def _finish(loss_ref, cnt_ref, out_ref):
    ls = jnp.sum(loss_ref[...])
    nv = jnp.sum(cnt_ref[...])
    out_ref[...] = (ls / (nv + 1e-6)).reshape(1, 1)


def kernel(curr_heightmap, new_roi, mask):
    B, _, H, W = curr_heightmap.shape
    HW = H * W
    n_chunks = 2
    Bc = B // n_chunks
    sc_call = _make_sc_partials(Bc, H, W)
    parts = []
    for h in range(n_chunks):
        sl = slice(h * Bc, (h + 1) * Bc)
        curr2 = curr_heightmap[sl].reshape(Bc, HW)
        roi2 = new_roi[sl].reshape(Bc, 3, HW)
        mask2 = mask[sl].reshape(Bc, HW)
        parts.append(sc_call(curr2, roi2, mask2))
    loss_p = jnp.concatenate([p[0] for p in parts])
    cnt_p = jnp.concatenate([p[1] for p in parts])
    out = pl.pallas_call(
        _finish,
        out_shape=jax.ShapeDtypeStruct((1, 1), jnp.float32),
    )(loss_p, cnt_p)
    return out[0, 0]


# speedup vs baseline: 1.3765x; 1.2028x over previous
import functools

import jax
import jax.numpy as jnp
from jax import lax
from jax.experimental import pallas as pl
from jax.experimental.pallas import tpu as pltpu
from jax.experimental.pallas import tpu_sc as plsc

_NC = 2
_NS = 16
_NW = _NC * _NS
_L = 16


def _make_sc_partials(B, H, W, CHW):
    HW = H * W
    assert B % (2 * _NW) == 0 and HW % _L == 0 and CHW % _L == 0
    bpw = B // _NW
    n_steps = CHW // _L
    mesh = plsc.VectorSubcoreMesh(core_axis_name="c", subcore_axis_name="s")

    @functools.partial(
        pl.kernel,
        mesh=mesh,
        compiler_params=pltpu.CompilerParams(needs_layout_passes=False),
        out_type=[
            jax.ShapeDtypeStruct((_NW, _L), jnp.float32),
            jax.ShapeDtypeStruct((_NW, _L), jnp.float32),
        ],
        scratch_types=[
            pltpu.VMEM((CHW,), jnp.float32),
            pltpu.VMEM((CHW,), jnp.float32),
            pltpu.VMEM((CHW,), jnp.float32),
            pltpu.VMEM((CHW,), jnp.float32),
            pltpu.VMEM((CHW,), jnp.float32),
            pltpu.VMEM((CHW,), jnp.float32),
            pltpu.VMEM((HW,), jnp.float32),
            pltpu.VMEM((HW,), jnp.float32),
            pltpu.VMEM((HW,), jnp.float32),
            pltpu.VMEM((HW,), jnp.float32),
            pltpu.VMEM((_L,), jnp.float32),
            pltpu.VMEM((_L,), jnp.float32),
            pltpu.SemaphoreType.DMA,
            pltpu.SemaphoreType.DMA,
        ],
    )
    def sc_kernel(curr_hbm, mask_hbm, y_hbm, x_hbm, t_hbm, loss_out, cnt_out,
                  y_v0, y_v1, x_v0, x_v1, t_v0, t_v1,
                  curr_v0, curr_v1, mask_v0, mask_v1,
                  loss_v, cnt_v, sem0, sem1):
        wid = lax.axis_index("s") * _NC + lax.axis_index("c")
        base = wid * bpw
        bufs = ((y_v0, x_v0, t_v0, curr_v0, mask_v0, sem0),
                (y_v1, x_v1, t_v1, curr_v1, mask_v1, sem1))

        def fire(b, k):
            y_v, x_v, t_v, curr_v, mask_v, sem = bufs[k]
            pltpu.make_async_copy(curr_hbm.at[b], curr_v, sem).start()
            pltpu.make_async_copy(mask_hbm.at[b], mask_v, sem).start()
            pltpu.make_async_copy(y_hbm.at[b], y_v, sem).start()
            pltpu.make_async_copy(x_hbm.at[b], x_v, sem).start()
            pltpu.make_async_copy(t_hbm.at[b], t_v, sem).start()

        def drain(b, k):
            y_v, x_v, t_v, curr_v, mask_v, sem = bufs[k]
            pltpu.make_async_copy(curr_hbm.at[b], curr_v, sem).wait()
            pltpu.make_async_copy(mask_hbm.at[b], mask_v, sem).wait()
            pltpu.make_async_copy(y_hbm.at[b], y_v, sem).wait()
            pltpu.make_async_copy(x_hbm.at[b], x_v, sem).wait()
            pltpu.make_async_copy(t_hbm.at[b], t_v, sem).wait()

        def compute(k, accs):
            y_v, x_v, t_v, curr_v, mask_v, _ = bufs[k]

            def step(j, accs2):
                lacc, cacc = accs2
                sl = pl.ds(j * _L, _L)
                y = y_v[sl].astype(jnp.int32)
                x = x_v[sl].astype(jnp.int32)
                t = t_v[sl]
                valid = (lax.bitcast_convert_type(y, jnp.uint32) < H) & (
                    lax.bitcast_convert_type(x, jnp.uint32) < W)
                flat = jnp.minimum(jnp.maximum(y * W + x, 0), HW - 1)
                c = plsc.load_gather(curr_v, [flat])
                m = plsc.load_gather(mask_v, [flat])
                d = c - t
                ad = jnp.abs(d)
                loss = jnp.where(ad < 1.0, 0.5 * d * d, ad - 0.5)
                w = jnp.where(valid, m, 0.0)
                return (lacc + loss * w, cacc + w)

            return lax.fori_loop(0, n_steps, step, accs, unroll=8)

        fire(base, 0)

        def pair_body(ip, accs):
            for k in (0, 1):
                i = 2 * ip + k
                b = base + i

                @pl.when(i + 1 < bpw)
                def _():
                    fire(b + 1, 1 - k)

                drain(b, k)
                accs = compute(k, accs)
            return accs

        zero = jnp.zeros((_L,), jnp.float32)
        lacc, cacc = lax.fori_loop(0, bpw // 2, pair_body, (zero, zero))
        loss_v[...] = lacc
        cnt_v[...] = cacc
        pltpu.sync_copy(loss_v, loss_out.at[wid])
        pltpu.sync_copy(cnt_v, cnt_out.at[wid])

    return sc_kernel


def _finish(refs):
    *ins, out_ref = refs
    ls = jnp.float32(0)
    nv = jnp.float32(0)
    for i in range(0, len(ins), 2):
        ls = ls + jnp.sum(ins[i][...])
        nv = nv + jnp.sum(ins[i + 1][...])
    out_ref[...] = (ls / (nv + 1e-6)).reshape(1, 1)


def kernel(curr_heightmap, new_roi, mask):
    B, _, H, W = curr_heightmap.shape
    HW = H * W
    n_chunks = 2
    CH = H // n_chunks
    CHW = CH * W
    curr2 = curr_heightmap.reshape(B, HW)
    mask2 = mask.reshape(B, HW)
    sc_call = _make_sc_partials(B, H, W, CHW)
    parts = []
    for k in range(n_chunks):
        rows = slice(k * CH, (k + 1) * CH)
        yk = new_roi[:, 0, rows, :].reshape(B, CHW)
        xk = new_roi[:, 1, rows, :].reshape(B, CHW)
        tk = new_roi[:, 2, rows, :].reshape(B, CHW)
        parts.extend(sc_call(curr2, mask2, yk, xk, tk))
    out = pl.pallas_call(
        lambda *refs: _finish(refs),
        out_shape=jax.ShapeDtypeStruct((1, 1), jnp.float32),
    )(*parts)
    return out[0, 0]


# speedup vs baseline: 1.7337x; 1.2595x over previous
import functools

import jax
import jax.numpy as jnp
from jax import lax
from jax.experimental import pallas as pl
from jax.experimental.pallas import tpu as pltpu
from jax.experimental.pallas import tpu_sc as plsc

_NC = 2
_NS = 16
_NW = _NC * _NS
_L = 16


def _make_sc_partials(B, H, W, CHW):
    HW = H * W
    assert B % (2 * _NW) == 0 and HW % _L == 0 and CHW % _L == 0
    bpw = B // _NW
    n_steps = CHW // _L
    mesh = plsc.VectorSubcoreMesh(core_axis_name="c", subcore_axis_name="s")

    @functools.partial(
        pl.kernel,
        mesh=mesh,
        compiler_params=pltpu.CompilerParams(needs_layout_passes=False),
        out_type=[
            jax.ShapeDtypeStruct((_NW, _L), jnp.float32),
            jax.ShapeDtypeStruct((_NW, _L), jnp.float32),
        ],
        scratch_types=[
            pltpu.VMEM((CHW,), jnp.float32),
            pltpu.VMEM((CHW,), jnp.float32),
            pltpu.VMEM((CHW,), jnp.float32),
            pltpu.VMEM((CHW,), jnp.float32),
            pltpu.VMEM((CHW,), jnp.float32),
            pltpu.VMEM((CHW,), jnp.float32),
            pltpu.VMEM((HW,), jnp.float32),
            pltpu.VMEM((HW,), jnp.float32),
            pltpu.VMEM((HW,), jnp.float32),
            pltpu.VMEM((HW,), jnp.float32),
            pltpu.VMEM((_L,), jnp.float32),
            pltpu.VMEM((_L,), jnp.float32),
            pltpu.SemaphoreType.DMA,
            pltpu.SemaphoreType.DMA,
        ],
    )
    def sc_kernel(curr_hbm, mask_hbm, y_hbm, x_hbm, t_hbm, loss_out, cnt_out,
                  y_v0, y_v1, x_v0, x_v1, t_v0, t_v1,
                  curr_v0, curr_v1, mask_v0, mask_v1,
                  loss_v, cnt_v, sem0, sem1):
        wid = lax.axis_index("s") * _NC + lax.axis_index("c")
        base = wid * bpw
        bufs = ((y_v0, x_v0, t_v0, curr_v0, mask_v0, sem0),
                (y_v1, x_v1, t_v1, curr_v1, mask_v1, sem1))

        def fire(b, k):
            y_v, x_v, t_v, curr_v, mask_v, sem = bufs[k]
            pltpu.make_async_copy(curr_hbm.at[b], curr_v, sem).start()
            pltpu.make_async_copy(mask_hbm.at[b], mask_v, sem).start()
            pltpu.make_async_copy(y_hbm.at[b], y_v, sem).start()
            pltpu.make_async_copy(x_hbm.at[b], x_v, sem).start()
            pltpu.make_async_copy(t_hbm.at[b], t_v, sem).start()

        def drain(b, k):
            y_v, x_v, t_v, curr_v, mask_v, sem = bufs[k]
            pltpu.make_async_copy(curr_hbm.at[b], curr_v, sem).wait()
            pltpu.make_async_copy(mask_hbm.at[b], mask_v, sem).wait()
            pltpu.make_async_copy(y_hbm.at[b], y_v, sem).wait()
            pltpu.make_async_copy(x_hbm.at[b], x_v, sem).wait()
            pltpu.make_async_copy(t_hbm.at[b], t_v, sem).wait()

        def compute(k, accs):
            y_v, x_v, t_v, curr_v, mask_v, _ = bufs[k]
            n_acc = 4

            def one(j):
                sl = pl.ds(j * _L, _L)
                y = y_v[sl].astype(jnp.int32)
                x = x_v[sl].astype(jnp.int32)
                t = t_v[sl]
                valid = (lax.bitcast_convert_type(y, jnp.uint32) < H) & (
                    lax.bitcast_convert_type(x, jnp.uint32) < W)
                flat_u = lax.bitcast_convert_type(y * W + x, jnp.uint32)
                flat = lax.bitcast_convert_type(
                    jnp.minimum(flat_u, jnp.uint32(HW - 1)), jnp.int32)
                c = plsc.load_gather(curr_v, [flat])
                m = plsc.load_gather(mask_v, [flat])
                d = c - t
                ad = jnp.abs(d)
                loss = jnp.where(ad < 1.0, 0.5 * d * d, ad - 0.5)
                w = jnp.where(valid, m, 0.0)
                return loss * w, w

            def group(g, accs2):
                out = []
                for q in range(n_acc):
                    lacc, cacc = accs2[q]
                    lw, w = one(g * n_acc + q)
                    out.append((lacc + lw, cacc + w))
                return tuple(out)

            return lax.fori_loop(0, n_steps // n_acc, group, accs, unroll=2)

        fire(base, 0)

        def pair_body(ip, accs):
            for k in (0, 1):
                i = 2 * ip + k
                b = base + i

                @pl.when(i + 1 < bpw)
                def _():
                    fire(b + 1, 1 - k)

                drain(b, k)
                accs = compute(k, accs)
            return accs

        zero = jnp.zeros((_L,), jnp.float32)
        accs0 = tuple((zero, zero) for _ in range(4))
        accs = lax.fori_loop(0, bpw // 2, pair_body, accs0)
        lacc = accs[0][0] + accs[1][0] + (accs[2][0] + accs[3][0])
        cacc = accs[0][1] + accs[1][1] + (accs[2][1] + accs[3][1])
        loss_v[...] = lacc
        cnt_v[...] = cacc
        pltpu.sync_copy(loss_v, loss_out.at[wid])
        pltpu.sync_copy(cnt_v, cnt_out.at[wid])

    return sc_kernel


def _finish(refs):
    *ins, out_ref = refs
    ls = jnp.float32(0)
    nv = jnp.float32(0)
    for i in range(0, len(ins), 2):
        ls = ls + jnp.sum(ins[i][...])
        nv = nv + jnp.sum(ins[i + 1][...])
    out_ref[...] = (ls / (nv + 1e-6)).reshape(1, 1)


def kernel(curr_heightmap, new_roi, mask):
    B, _, H, W = curr_heightmap.shape
    HW = H * W
    n_chunks = 2
    CH = H // n_chunks
    CHW = CH * W
    curr2 = curr_heightmap.reshape(B, HW)
    mask2 = mask.reshape(B, HW)
    sc_call = _make_sc_partials(B, H, W, CHW)
    parts = []
    for k in range(n_chunks):
        rows = slice(k * CH, (k + 1) * CH)
        yk = new_roi[:, 0, rows, :].reshape(B, CHW)
        xk = new_roi[:, 1, rows, :].reshape(B, CHW)
        tk = new_roi[:, 2, rows, :].reshape(B, CHW)
        parts.extend(sc_call(curr2, mask2, yk, xk, tk))
    out = pl.pallas_call(
        lambda *refs: _finish(refs),
        out_shape=jax.ShapeDtypeStruct((1, 1), jnp.float32),
    )(*parts)
    return out[0, 0]
